# baseline (speedup 1.0000x reference)
import jax
import jax.numpy as jnp
import numpy as np
from jax.experimental import pallas as pl

B = 1
S = 2048
HID = 2048
NH = 16
QLORA = 1536
KVLORA = 512
NOPE = 128
ROPE = 64
QKD = NOPE + ROPE
VD = 128
IH = 8
ID = 128
TOPK = 512

BQ2 = 256


def _hadamard(n):
    H = np.array([[1.0]], dtype=np.float32)
    while H.shape[0] < n:
        H = np.block([[H, H], [H, -H]])
    return H

_HMAT = jnp.asarray(_hadamard(ID) * (ID ** -0.5), dtype=jnp.bfloat16)


def _rotate_activation(x):
    return x.astype(jnp.bfloat16) @ _HMAT


def _rms_norm(x, w, eps=1e-6):
    v = jnp.mean(jnp.square(x), axis=-1, keepdims=True)
    return x * jax.lax.rsqrt(v + eps) * w


def _layer_norm(x, w, b, eps=1e-5):
    m = jnp.mean(x, axis=-1, keepdims=True)
    v = jnp.var(x, axis=-1, keepdims=True)
    return (x - m) * jax.lax.rsqrt(v + eps) * w + b


def _rope4(x, fc):
    b, s, h, d = x.shape
    xr = x.reshape(b, s, h, d // 2, 2)
    cos = fc[None, :, None, :, 0]
    sin = fc[None, :, None, :, 1]
    o1 = xr[..., 0] * cos - xr[..., 1] * sin
    o2 = xr[..., 0] * sin + xr[..., 1] * cos
    return jnp.stack([o1, o2], axis=-1).reshape(b, s, h, d)


def _rope3(x, fc):
    b, s, d = x.shape
    xr = x.reshape(b, s, d // 2, 2)
    cos = fc[None, :, :, 0]
    sin = fc[None, :, :, 1]
    o1 = xr[..., 0] * cos - xr[..., 1] * sin
    o2 = xr[..., 0] * sin + xr[..., 1] * cos
    return jnp.stack([o1, o2], axis=-1).reshape(b, s, d)


def _attn_kernel(q_ref, k_ref, v_ref, mask_ref, o_ref):
    q = q_ref[0]
    k = k_ref[0]
    v = v_ref[0]
    s = jax.lax.dot_general(q, k, (((1,), (1,)), ((), ())),
                            preferred_element_type=jnp.float32,
                            precision=jax.lax.Precision.HIGHEST)
    s = s * jnp.float32(QKD ** -0.5)
    s = jnp.where(mask_ref[...] > 0, s, jnp.float32(-1e30))
    mx = jnp.max(s, axis=1, keepdims=True)
    p = jnp.exp(s - mx)
    d = jnp.sum(p, axis=1, keepdims=True)
    o = jax.lax.dot_general(p, v, (((1,), (0,)), ((), ())),
                            preferred_element_type=jnp.float32,
                            precision=jax.lax.Precision.HIGHEST)
    o_ref[0] = o / d


def kernel(x, freqs_cis, Wqa, qa_ln_w, Wqb, Wkva, kva_ln_w, Wkvb, Wo, Wiq,
           Wik, ik_ln_w, ik_ln_b, Wiw):
    b, s, _ = x.shape
    q_resid = _rms_norm(x @ Wqa, qa_ln_w)
    qi = (q_resid @ Wiq).reshape(b, s, IH, ID)
    qi_nope, qi_pe = qi[..., : ID - ROPE], qi[..., ID - ROPE:]
    ki = _layer_norm(x @ Wik, ik_ln_w, ik_ln_b)
    ki_nope, ki_pe = ki[..., : ID - ROPE], ki[..., ID - ROPE:]
    qi_pe = _rope4(qi_pe, freqs_cis)
    ki_pe = _rope3(ki_pe, freqs_cis)
    qi = jnp.concatenate([qi_nope, qi_pe], axis=-1)
    ki = jnp.concatenate([ki_nope, ki_pe], axis=-1)
    qi = _rotate_activation(qi).astype(jnp.float32)
    ki = _rotate_activation(ki).astype(jnp.float32)
    wts = (x @ Wiw).astype(jnp.float32) * (IH ** -0.5) * (ID ** -0.5)
    iscores = jnp.einsum('bshd,btd->bhst', qi, ki)
    iscores = jax.nn.relu(iscores) * jnp.transpose(wts, (0, 2, 1))[..., None]
    iscores = jnp.sum(iscores, axis=1)
    topk = min(TOPK, s)
    _, topk_idx = jax.lax.top_k(iscores, topk)

    q = (q_resid @ Wqb).reshape(b, s, NH, QKD)
    q_nope, q_pe = q[..., :NOPE], q[..., NOPE:]
    kv = x @ Wkva
    c_kv, k_pe = kv[..., :KVLORA], kv[..., KVLORA:]
    c_kv = _rms_norm(c_kv, kva_ln_w)
    kvb = (c_kv @ Wkvb).reshape(b, s, NH, NOPE + VD)
    k_nope, v = kvb[..., :NOPE], kvb[..., NOPE:]
    q_pe = _rope4(q_pe, freqs_cis)
    k_pe = _rope3(k_pe, freqs_cis)
    q = jnp.concatenate([q_nope, q_pe], axis=-1)
    k = jnp.concatenate(
        [k_nope, jnp.broadcast_to(k_pe[:, :, None, :], (b, s, NH, ROPE))],
        axis=-1)

    bi = jnp.arange(b)[:, None, None]
    si = jnp.arange(s)[None, :, None]
    mask = jnp.zeros((b, s, s), dtype=bool).at[bi, si, topk_idx].set(True)

    qT = q[0].transpose(1, 0, 2)
    kT = k[0].transpose(1, 0, 2)
    vT = v[0].transpose(1, 0, 2)

    attn = pl.pallas_call(
        _attn_kernel,
        grid=(NH, s // BQ2),
        in_specs=[
            pl.BlockSpec((1, BQ2, QKD), lambda h, i: (h, i, 0)),
            pl.BlockSpec((1, s, QKD), lambda h, i: (h, 0, 0)),
            pl.BlockSpec((1, s, VD), lambda h, i: (h, 0, 0)),
            pl.BlockSpec((BQ2, s), lambda h, i: (i, 0)),
        ],
        out_specs=pl.BlockSpec((1, BQ2, VD), lambda h, i: (h, i, 0)),
        out_shape=jax.ShapeDtypeStruct((NH, s, VD), jnp.float32),
    )(qT, kT, vT, mask[0].astype(jnp.bfloat16))

    out = attn.transpose(1, 0, 2).reshape(b, s, NH * VD)
    return out @ Wo
